# baseline (speedup 1.0000x reference)
import functools

import jax
import jax.numpy as jnp
from jax import lax
from jax.experimental import pallas as pl
from jax.experimental.pallas import tpu as pltpu
from jax.experimental.pallas import tpu_sc as plsc

E = 100000
S = 128
NX = 256
NY = 256
DX = 2.0
DY = 2.0
TIME_RES = 400.0
C_MM_PER_PS = 0.299792458
FWHM_TO_SIGMA = 2.3548200450309493

NC = 2
NS = 16
NW = NC * NS
GROUPS_PER_W = 196
EV_PER_W = GROUPS_PER_W * 16

PAD = 24
PX_DIM = NX + 2 * PAD
PY_DIM = NY + 2 * PAD
IMG_WORDS = PX_DIM * PY_DIM

SIGMA = TIME_RES * C_MM_PER_PS * 0.5 / FWHM_TO_SIGMA
Q = 1.0 / (SIGMA * (2.0 ** 0.5))
HALF_C = C_MM_PER_PS * 0.5
DT = 1.0 / (S - 1)


def _rsqrt(x):
    xi = lax.bitcast_convert_type(x, jnp.int32)
    yi = jnp.int32(0x5F3759DF) - (xi >> 1)
    y = lax.bitcast_convert_type(yi, jnp.float32)
    for _ in range(3):
        y = y * (1.5 - 0.5 * x * y * y)
    return y


def _sc_project(img, tof, x1l, y1l, x1r, y1r, x2l, y2l, x2r, y2r):
    mesh = plsc.VectorSubcoreMesh(core_axis_name="c", subcore_axis_name="s")

    @functools.partial(
        pl.kernel,
        mesh=mesh,
        out_type=jax.ShapeDtypeStruct((E,), jnp.float32),
        compiler_params=pltpu.CompilerParams(needs_layout_passes=False),
        scratch_types=[pltpu.VMEM((IMG_WORDS,), jnp.float32)]
        + [pltpu.VMEM((EV_PER_W,), jnp.float32) for _ in range(10)]
        + [pltpu.SemaphoreType.DMA],
    )
    def run(img_h, tof_h, x1l_h, y1l_h, x1r_h, y1r_h, x2l_h, y2l_h,
            x2r_h, y2r_h, out_h, img_v, tof_v, x1l_v, y1l_v, x1r_v,
            y1r_v, x2l_v, y2l_v, x2r_v, y2r_v, out_v, dma_sem):
        wid = lax.axis_index("s") * NC + lax.axis_index("c")
        base = jnp.minimum(wid * EV_PER_W, E - EV_PER_W)
        copies = [pltpu.async_copy(img_h, img_v, dma_sem)]
        for h, v in ((tof_h, tof_v), (x1l_h, x1l_v), (y1l_h, y1l_v),
                     (x1r_h, x1r_v), (y1r_h, y1r_v), (x2l_h, x2l_v),
                     (y2l_h, y2l_v), (x2r_h, x2r_v), (y2r_h, y2r_v)):
            copies.append(
                pltpu.async_copy(h.at[pl.ds(base, EV_PER_W)], v, dma_sem))
        for cp in copies:
            cp.wait()

        def group(g, carry):
            off = g * 16
            x1 = 0.5 * (x1l_v[pl.ds(off, 16)] + x1r_v[pl.ds(off, 16)])
            y1 = 0.5 * (y1l_v[pl.ds(off, 16)] + y1r_v[pl.ds(off, 16)])
            x2 = 0.5 * (x2l_v[pl.ds(off, 16)] + x2r_v[pl.ds(off, 16)])
            y2 = 0.5 * (y2l_v[pl.ds(off, 16)] + y2r_v[pl.ds(off, 16)])
            tofv = tof_v[pl.ds(off, 16)]
            ddx = x2 - x1
            ddy = y2 - y1
            l2 = ddx * ddx + ddy * ddy
            rl = _rsqrt(l2)
            ln = l2 * rl
            fx0 = x1 * (1.0 / DX) + (NX * 0.5 - 0.5 + PAD)
            dfx = ddx * (DT / DX)
            fy0 = y1 * (1.0 / DY) + (NY * 0.5 - 0.5 + PAD)
            dfy = ddy * (DT / DY)
            zz0 = -(0.5 * ln + tofv * HALF_C) * Q
            dzz = ln * (DT * Q)
            scale = ln * DT

            def sample(_, c):
                acc, fx, fy, zz = c
                w = jnp.exp(-(zz * zz))
                ixf = fx.astype(jnp.int32)
                iyf = fy.astype(jnp.int32)
                wx = fx - ixf.astype(jnp.float32)
                wy = fy - iyf.astype(jnp.float32)
                idx = ixf * PY_DIM + iyf
                idx1 = idx + 1
                row1 = img_v.at[pl.ds(PY_DIM, IMG_WORDS - PY_DIM)]
                v00 = plsc.load_gather(img_v, [idx])
                v01 = plsc.load_gather(img_v, [idx1])
                v10 = plsc.load_gather(row1, [idx])
                v11 = plsc.load_gather(row1, [idx1])
                vx0 = v00 + wx * (v10 - v00)
                vx1 = v01 + wx * (v11 - v01)
                val = vx0 + wy * (vx1 - vx0)
                return (acc + val * w, fx + dfx, fy + dfy, zz + dzz)

            acc, _, _, _ = lax.fori_loop(
                0, S, sample,
                (jnp.zeros((16,), jnp.float32), fx0, fy0, zz0),
                unroll=8)
            out_v[pl.ds(off, 16)] = acc * scale
            return carry

        lax.fori_loop(0, GROUPS_PER_W, group, 0)
        pltpu.sync_copy(out_v, out_h.at[pl.ds(base, EV_PER_W)])

    return run(img, tof, x1l, y1l, x1r, y1r, x2l, y2l, x2r, y2r)


def kernel(image, tof_value, x1l, y1l, x1r, y1r, x2l, y2l, x2r, y2r):
    imgp = jnp.zeros((PX_DIM, PY_DIM), jnp.float32)
    imgp = imgp.at[PAD:PAD + NX, PAD:PAD + NY].set(image)
    return _sc_project(imgp.reshape(-1), tof_value, x1l, y1l, x1r, y1r,
                       x2l, y2l, x2r, y2r)


# speedup vs baseline: 1.0337x; 1.0337x over previous
import functools

import jax
import jax.numpy as jnp
from jax import lax
from jax.experimental import pallas as pl
from jax.experimental.pallas import tpu as pltpu
from jax.experimental.pallas import tpu_sc as plsc

E = 100000
S = 128
NX = 256
NY = 256
DX = 2.0
DY = 2.0
TIME_RES = 400.0
C_MM_PER_PS = 0.299792458
FWHM_TO_SIGMA = 2.3548200450309493

NC = 2
NS = 16
NW = NC * NS
GROUPS_PER_W = 196
EV_PER_W = GROUPS_PER_W * 16

PAD = 24
PX_DIM = NX + 2 * PAD
PY_DIM = NY + 2 * PAD
IMG_WORDS = PX_DIM * PY_DIM

SIGMA = TIME_RES * C_MM_PER_PS * 0.5 / FWHM_TO_SIGMA
Q = 1.0 / (SIGMA * (2.0 ** 0.5))
HALF_C = C_MM_PER_PS * 0.5
DT = 1.0 / (S - 1)


def _rsqrt(x):
    xi = lax.bitcast_convert_type(x, jnp.int32)
    yi = jnp.int32(0x5F3759DF) - (xi >> 1)
    y = lax.bitcast_convert_type(yi, jnp.float32)
    for _ in range(3):
        y = y * (1.5 - 0.5 * x * y * y)
    return y


def _sc_project(img, tof, x1l, y1l, x1r, y1r, x2l, y2l, x2r, y2r):
    mesh = plsc.VectorSubcoreMesh(core_axis_name="c", subcore_axis_name="s")

    @functools.partial(
        pl.kernel,
        mesh=mesh,
        out_type=jax.ShapeDtypeStruct((E,), jnp.float32),
        compiler_params=pltpu.CompilerParams(needs_layout_passes=False),
        scratch_types=[pltpu.VMEM((IMG_WORDS,), jnp.float32)]
        + [pltpu.VMEM((EV_PER_W,), jnp.float32) for _ in range(10)]
        + [pltpu.SemaphoreType.DMA],
    )
    def run(img_h, tof_h, x1l_h, y1l_h, x1r_h, y1r_h, x2l_h, y2l_h,
            x2r_h, y2r_h, out_h, img_v, tof_v, x1l_v, y1l_v, x1r_v,
            y1r_v, x2l_v, y2l_v, x2r_v, y2r_v, out_v, dma_sem):
        wid = lax.axis_index("s") * NC + lax.axis_index("c")
        base = jnp.minimum(wid * EV_PER_W, E - EV_PER_W)
        copies = [pltpu.async_copy(img_h, img_v, dma_sem)]
        for h, v in ((tof_h, tof_v), (x1l_h, x1l_v), (y1l_h, y1l_v),
                     (x1r_h, x1r_v), (y1r_h, y1r_v), (x2l_h, x2l_v),
                     (y2l_h, y2l_v), (x2r_h, x2r_v), (y2r_h, y2r_v)):
            copies.append(
                pltpu.async_copy(h.at[pl.ds(base, EV_PER_W)], v, dma_sem))
        for cp in copies:
            cp.wait()

        def group(g, carry):
            off = g * 16
            x1 = 0.5 * (x1l_v[pl.ds(off, 16)] + x1r_v[pl.ds(off, 16)])
            y1 = 0.5 * (y1l_v[pl.ds(off, 16)] + y1r_v[pl.ds(off, 16)])
            x2 = 0.5 * (x2l_v[pl.ds(off, 16)] + x2r_v[pl.ds(off, 16)])
            y2 = 0.5 * (y2l_v[pl.ds(off, 16)] + y2r_v[pl.ds(off, 16)])
            tofv = tof_v[pl.ds(off, 16)]
            ddx = x2 - x1
            ddy = y2 - y1
            l2 = ddx * ddx + ddy * ddy
            rl = _rsqrt(l2)
            ln = l2 * rl
            fx0 = x1 * (1.0 / DX) + (NX * 0.5 - 0.5 + PAD)
            dfx = ddx * (DT / DX)
            fy0 = y1 * (1.0 / DY) + (NY * 0.5 - 0.5 + PAD)
            dfy = ddy * (DT / DY)
            zz0 = -(0.5 * ln + tofv * HALF_C) * Q
            dzz = ln * (DT * Q)
            scale = ln * DT

            def sample(_, c):
                acc, fx, fy, zz = c
                w = jnp.exp(-(zz * zz))
                ixf = fx.astype(jnp.int32)
                iyf = fy.astype(jnp.int32)
                wx = fx - ixf.astype(jnp.float32)
                wy = fy - iyf.astype(jnp.float32)
                idx = ixf * PY_DIM + iyf
                v00 = plsc.load_gather(img_v, [idx])
                v01 = plsc.load_gather(img_v, [idx + 1])
                v10 = plsc.load_gather(img_v, [idx + PY_DIM])
                v11 = plsc.load_gather(img_v, [idx + (PY_DIM + 1)])
                vx0 = v00 + wx * (v10 - v00)
                vx1 = v01 + wx * (v11 - v01)
                val = vx0 + wy * (vx1 - vx0)
                return (acc + val * w, fx + dfx, fy + dfy, zz + dzz)

            acc, _, _, _ = lax.fori_loop(
                0, S, sample,
                (jnp.zeros((16,), jnp.float32), fx0, fy0, zz0),
                unroll=8)
            out_v[pl.ds(off, 16)] = acc * scale
            return carry

        lax.fori_loop(0, GROUPS_PER_W, group, 0)
        pltpu.sync_copy(out_v, out_h.at[pl.ds(base, EV_PER_W)])

    return run(img, tof, x1l, y1l, x1r, y1r, x2l, y2l, x2r, y2r)


def kernel(image, tof_value, x1l, y1l, x1r, y1r, x2l, y2l, x2r, y2r):
    imgp = jnp.zeros((PX_DIM, PY_DIM), jnp.float32)
    imgp = imgp.at[PAD:PAD + NX, PAD:PAD + NY].set(image)
    return _sc_project(imgp.reshape(-1), tof_value, x1l, y1l, x1r, y1r,
                       x2l, y2l, x2r, y2r)
